# baseline (speedup 1.0000x reference)
import functools

import jax
import jax.numpy as jnp
from jax import lax
from jax.experimental import pallas as pl
from jax.experimental.pallas import tpu as pltpu
from jax.experimental.pallas import tpu_sc as plsc

N_NODES = 10000
E_EDGES = 160000
D_FEAT = 32
D_EDGE = 16
DIM = 32
ISO_N = 4096
N_ASSIGN = 16384
E2 = 65536
NUM_I2 = 32
NHEADS = 4
NGRAPH = 128

NC = 2
NS = 16
NW = NC * NS
L = 16
CH = 128

E_PAD = ((E_EDGES + NW * CH - 1) // (NW * CH)) * (NW * CH)
AGG_ROWS = N_NODES + 16

_SC_PARAMS = pltpu.CompilerParams(use_tc_tiling_on_sc=False)


def _sc_mesh():
  return plsc.VectorSubcoreMesh(core_axis_name="c", subcore_axis_name="s",
                                num_cores=NC, num_subcores=NS)


def _sc_gather(table, idx):
  V, D = table.shape
  B = idx.shape[0]
  n_ch = B // (NW * CH)

  @functools.partial(
      pl.kernel, mesh=_sc_mesh(), compiler_params=_SC_PARAMS,
      out_type=jax.ShapeDtypeStruct((B, D), jnp.float32),
      scratch_types=[
          pltpu.VMEM((CH,), jnp.int32),
          pltpu.VMEM((CH, D), jnp.float32),
          pltpu.SemaphoreType.DMA,
      ])
  def k(table_h, idx_h, out_h, idx_v, rows_v, sem):
    wid = lax.axis_index("s") * NC + lax.axis_index("c")

    def body(j, carry):
      off = (wid * n_ch + j) * CH
      pltpu.sync_copy(idx_h.at[pl.ds(off, CH)], idx_v)
      pltpu.async_copy(table_h.at[idx_v], rows_v, sem).wait()
      pltpu.sync_copy(rows_v, out_h.at[pl.ds(off, CH)])
      return carry

    lax.fori_loop(0, n_ch, body, 0)

  return k(table, idx)


def _sc_scatter_sum(msg, dst, zeros):
  B, D = msg.shape
  zch = zeros.shape[0]
  per_core = B // NC
  n_ch = per_core // (NS * CH)
  rows_pt = AGG_ROWS // NS

  @functools.partial(
      pl.kernel, mesh=_sc_mesh(), compiler_params=_SC_PARAMS,
      out_type=jax.ShapeDtypeStruct((NC * AGG_ROWS, D), jnp.float32),
      scratch_types=[
          pltpu.VMEM((CH,), jnp.int32),
          pltpu.VMEM((CH, D), jnp.float32),
          pltpu.VMEM((zch, D), jnp.float32),
          pltpu.VMEM_SHARED((AGG_ROWS, D), jnp.float32),
          pltpu.SemaphoreType.DMA,
      ])
  def k(msg_h, dst_h, zeros_h, out_h, idx_v, rows_v, zb_v, acc_s, sem):
    c = lax.axis_index("c")
    s = lax.axis_index("s")
    pltpu.sync_copy(zeros_h, zb_v)
    done = 0
    while done < rows_pt:
      m = min(zch, rows_pt - done)
      pltpu.sync_copy(zb_v.at[pl.ds(0, m)], acc_s.at[pl.ds(s * rows_pt + done, m)])
      done += m
    plsc.subcore_barrier()

    def body(j, carry):
      off = c * per_core + (s * n_ch + j) * CH
      pltpu.sync_copy(dst_h.at[pl.ds(off, CH)], idx_v)
      pltpu.sync_copy(msg_h.at[pl.ds(off, CH)], rows_v)
      pltpu.sync_copy(rows_v, acc_s.at[idx_v], add=True)
      return carry

    lax.fori_loop(0, n_ch, body, 0)
    plsc.subcore_barrier()
    done = 0
    while done < rows_pt:
      m = min(zch, rows_pt - done)
      r0 = s * rows_pt + done
      pltpu.sync_copy(acc_s.at[pl.ds(r0, m)], zb_v.at[pl.ds(0, m)])
      pltpu.sync_copy(zb_v.at[pl.ds(0, m)], out_h.at[pl.ds(c * AGG_ROWS + r0, m)])
      done += m

  return k(msg, dst, zeros)


def _sc_assign_mean(table, a_src, a_dst, zeros32, ones16):
  V, D = table.shape
  B = a_src.shape[0]
  per_core = B // NC
  n_ch = per_core // (NS * CH)
  rows_pt = ISO_N // NS
  zch = zeros32.shape[0]

  @functools.partial(
      pl.kernel, mesh=_sc_mesh(), compiler_params=_SC_PARAMS,
      out_type=(jax.ShapeDtypeStruct((NC * ISO_N, D), jnp.float32),
                jax.ShapeDtypeStruct((NC * ISO_N, 16), jnp.float32)),
      scratch_types=[
          pltpu.VMEM((CH,), jnp.int32),
          pltpu.VMEM((CH,), jnp.int32),
          pltpu.VMEM((CH, D), jnp.float32),
          pltpu.VMEM((CH, 16), jnp.float32),
          pltpu.VMEM((zch, D), jnp.float32),
          pltpu.VMEM_SHARED((ISO_N, D), jnp.float32),
          pltpu.VMEM_SHARED((ISO_N, 16), jnp.float32),
          pltpu.SemaphoreType.DMA,
      ])
  def k(table_h, src_h, dst_h, zeros_h, ones_h, sum_h, cnt_h,
        si_v, di_v, rows_v, ones_v, zb_v, sum_s, cnt_s, sem):
    c = lax.axis_index("c")
    s = lax.axis_index("s")
    pltpu.sync_copy(zeros_h, zb_v)
    pltpu.sync_copy(ones_h, ones_v)
    done = 0
    while done < rows_pt:
      m = min(zch, rows_pt - done)
      r0 = s * rows_pt + done
      pltpu.sync_copy(zb_v.at[pl.ds(0, m)], sum_s.at[pl.ds(r0, m)])
      pltpu.sync_copy(zb_v.at[pl.ds(0, m), pl.ds(0, 16)], cnt_s.at[pl.ds(r0, m)])
      done += m
    plsc.subcore_barrier()

    def body(j, carry):
      off = c * per_core + (s * n_ch + j) * CH
      pltpu.sync_copy(src_h.at[pl.ds(off, CH)], si_v)
      pltpu.sync_copy(dst_h.at[pl.ds(off, CH)], di_v)
      pltpu.async_copy(table_h.at[si_v], rows_v, sem).wait()
      pltpu.sync_copy(rows_v, sum_s.at[di_v], add=True)
      pltpu.sync_copy(ones_v, cnt_s.at[di_v], add=True)
      return carry

    lax.fori_loop(0, n_ch, body, 0)
    plsc.subcore_barrier()
    done = 0
    while done < rows_pt:
      m = min(zch, rows_pt - done)
      r0 = s * rows_pt + done
      pltpu.sync_copy(sum_s.at[pl.ds(r0, m)], zb_v.at[pl.ds(0, m)])
      pltpu.sync_copy(zb_v.at[pl.ds(0, m)], sum_h.at[pl.ds(c * ISO_N + r0, m)])
      pltpu.sync_copy(cnt_s.at[pl.ds(r0, m)], zb_v.at[pl.ds(0, m), pl.ds(0, 16)])
      pltpu.sync_copy(zb_v.at[pl.ds(0, m), pl.ds(0, 16)],
                      cnt_h.at[pl.ds(c * ISO_N + r0, m)])
      done += m

  return k(table, a_src, a_dst, zeros32, ones16)


ADJ_TOT = ISO_N * ISO_N + 64


def _sc_adj(e0, e1, zeros_flat, ones_flat):
  half = (ISO_N * ISO_N) // NC
  per_tile = E2 // NS
  n_ch = per_tile // CH
  zch = zeros_flat.shape[0]
  zrows = half // NS
  n_z = zrows // zch

  @functools.partial(
      pl.kernel, mesh=_sc_mesh(), compiler_params=_SC_PARAMS,
      out_type=jax.ShapeDtypeStruct((ADJ_TOT,), jnp.float32),
      scratch_types=[
          pltpu.VMEM((per_tile,), jnp.int32),
          pltpu.VMEM((per_tile,), jnp.int32),
          pltpu.VMEM((CH,), jnp.int32),
          pltpu.VMEM((CH,), jnp.float32),
          pltpu.VMEM((zch,), jnp.float32),
          pltpu.SemaphoreType.DMA,
      ])
  def k(e0_h, e1_h, zeros_h, ones_h, out_h, e0_v, e1_v, key_v, ones_v, zb_v, sem):
    c = lax.axis_index("c")
    s = lax.axis_index("s")
    lo = c * half
    pltpu.sync_copy(zeros_h, zb_v)

    def zbody(j, carry):
      pltpu.sync_copy(zb_v, out_h.at[pl.ds(lo + s * zrows + j * zch, zch)])
      return carry

    lax.fori_loop(0, n_z, zbody, 0)
    plsc.subcore_barrier()

    pltpu.sync_copy(ones_h, ones_v)
    pltpu.sync_copy(e0_h.at[pl.ds(s * per_tile, per_tile)], e0_v)
    pltpu.sync_copy(e1_h.at[pl.ds(s * per_tile, per_tile)], e1_v)

    def body(j, carry):
      for t in range(CH // L):
        a = e0_v[pl.ds(j * CH + t * L, L)]
        b = e1_v[pl.ds(j * CH + t * L, L)]
        key = a * ISO_N + b
        inside = (key >= lo) & (key < lo + half)
        key_v[pl.ds(t * L, L)] = jnp.where(inside, key, ISO_N * ISO_N)
      pltpu.sync_copy(ones_v, out_h.at[key_v])
      return carry

    lax.fori_loop(0, n_ch, body, 0)

  return k(e0, e1, zeros_flat, ones_flat)


_TC_PARAMS = pltpu.CompilerParams(dimension_semantics=("arbitrary",))


def _dot(a, b, dn=None, hi=False):
  if dn is None:
    dn = (((a.ndim - 1,), (0,)), ((), ()))
  prec = lax.Precision.HIGHEST if hi else None
  return lax.dot_general(a, b, dn, precision=prec,
                         preferred_element_type=jnp.float32)


def _msg_body(ea_ref, hs_ref, w1_ref, b1_ref, w2_ref, b2_ref, r_ref, s_ref,
              out_ref):
  e = jnp.maximum(_dot(ea_ref[...], w1_ref[...]) + b1_ref[...], 0.0)
  we = _dot(e, w2_ref[...]) + b2_ref[...]
  hx = _dot(hs_ref[...], r_ref[...], hi=True)
  out_ref[...] = _dot(we * hx, s_ref[...], hi=True)


def _tc_messages(ea_pad, h_src, p):
  mi, mo = p['root'].shape
  blk = 2048
  grid = ea_pad.shape[0] // blk
  rmat = jnp.kron(jnp.eye(mi, dtype=jnp.float32),
                  jnp.ones((1, mo), jnp.float32))
  smat = jnp.kron(jnp.ones((mi, 1), jnp.float32),
                  jnp.eye(mo, dtype=jnp.float32))
  full = lambda a: pl.BlockSpec(a.shape, lambda i: (0,) * a.ndim)
  args = (ea_pad, h_src, p['nn_W1'], p['nn_b1'].reshape(1, -1), p['nn_W2'],
          p['nn_b2'].reshape(1, -1), rmat, smat)
  return pl.pallas_call(
      _msg_body,
      grid=(grid,),
      in_specs=[pl.BlockSpec((blk, D_EDGE), lambda i: (i, 0)),
                pl.BlockSpec((blk, mi), lambda i: (i, 0))] +
               [full(a) for a in args[2:]],
      out_specs=pl.BlockSpec((blk, mo), lambda i: (i, 0)),
      out_shape=jax.ShapeDtypeStruct((ea_pad.shape[0], mo), jnp.float32),
      compiler_params=_TC_PARAMS,
  )(*args)


def _upd_body(h_ref, agg_ref, root_ref, bias_ref, out_ref):
  agg = agg_ref[0] + agg_ref[1]
  out_ref[...] = jnp.maximum(_dot(h_ref[...], root_ref[...]) + bias_ref[...]
                             + agg, 0.0)


def _tc_update(h, agg2, p):
  mi, mo = p['root'].shape
  full = lambda a: pl.BlockSpec(a.shape, lambda: (0,) * a.ndim)
  args = (h, agg2, p['root'], p['bias'].reshape(1, -1))
  return pl.pallas_call(
      _upd_body,
      in_specs=[full(a) for a in args],
      out_specs=pl.BlockSpec((N_NODES, mo), lambda: (0, 0)),
      out_shape=jax.ShapeDtypeStruct((N_NODES, mo), jnp.float32),
  )(*args)


def _prep_body(sum_ref, cnt_ref, iso_ref, w_ref, out_ref):
  s = sum_ref[0] + sum_ref[1]
  c = cnt_ref[0] + cnt_ref[1]
  h2 = s / jnp.maximum(c[:, 0:1], 1.0)
  cat = jnp.concatenate([h2, iso_ref[...]], axis=1)
  for h in range(NHEADS):
    out_ref[:, h * DIM:(h + 1) * DIM] = _dot(cat, w_ref[h])


def _tc_gat_prep(sums, cnts, iso, wst):
  full = lambda a: pl.BlockSpec(a.shape, lambda: (0,) * a.ndim)
  args = (sums, cnts, iso, wst)
  return pl.pallas_call(
      _prep_body,
      in_specs=[full(a) for a in args],
      out_specs=pl.BlockSpec((ISO_N, NHEADS * DIM), lambda: (0, 0)),
      out_shape=jax.ShapeDtypeStruct((ISO_N, NHEADS * DIM), jnp.float32),
  )(*args)


def _gat_body(adj_ref, wh_ref, a1_ref, a2_ref, out_ref):
  i = pl.program_id(0)
  blk = adj_ref.shape[0]
  mask = adj_ref[...] > 0.0
  acc = jnp.zeros((blk, DIM), jnp.float32)
  for h in range(NHEADS):
    whh = wh_ref[:, h * DIM:(h + 1) * DIM]
    whb = wh_ref[pl.ds(i * blk, blk), h * DIM:(h + 1) * DIM]
    f1 = _dot(whb, a1_ref[h:h + 1, :], (((1,), (1,)), ((), ())))
    f2t = _dot(a2_ref[h:h + 1, :], whh, (((1,), (1,)), ((), ())))
    e = f1 + f2t
    e = jnp.where(e >= 0.0, e, 0.2 * e)
    e = jnp.where(mask, e, -9e15)
    m = jnp.max(e, axis=1, keepdims=True)
    pexp = jnp.exp(e - m)
    ssum = jnp.sum(pexp, axis=1, keepdims=True)
    acc = acc + _dot(pexp, whh) / ssum
  a = acc * (1.0 / NHEADS)
  out_ref[...] = jnp.where(a > 0.0, a, jnp.exp(a) - 1.0)


def _tc_gat(adj, wh, a1s, a2s):
  blk = 256
  grid = ISO_N // blk
  full = lambda a: pl.BlockSpec(a.shape, lambda i: (0,) * a.ndim)
  return pl.pallas_call(
      _gat_body,
      grid=(grid,),
      in_specs=[pl.BlockSpec((blk, ISO_N), lambda i: (i, 0)),
                full(wh), full(a1s), full(a2s)],
      out_specs=pl.BlockSpec((blk, DIM), lambda i: (i, 0)),
      out_shape=jax.ShapeDtypeStruct((ISO_N, DIM), jnp.float32),
      compiler_params=_TC_PARAMS,
  )(adj, wh, a1s, a2s)


def _seg_mean_mm(ids_row, data):
  gi = lax.broadcasted_iota(jnp.int32, (NGRAPH, 1), 0)
  m = (ids_row == gi).astype(jnp.float32)
  ssum = _dot(m, data, hi=True)
  cnt = jnp.sum(m, axis=1, keepdims=True)
  return ssum / jnp.maximum(cnt, 1.0)


def _tail_body(h_ref, b_ref, g_ref, b2_ref, w0_ref, c0_ref, w1_ref, c1_ref,
               w2_ref, c2_ref, out_ref):
  x1 = _seg_mean_mm(b_ref[...], h_ref[...])
  x2 = _seg_mean_mm(b2_ref[...], g_ref[...])
  z = jnp.concatenate([x1, x2], axis=1)
  z = jnp.maximum(_dot(z, w0_ref[...]) + c0_ref[...], 0.0)
  z = jnp.maximum(_dot(z, w1_ref[...]) + c1_ref[...], 0.0)
  out_ref[...] = _dot(z, w2_ref[...]) + c2_ref[...]


def _tc_tail(h_conv, batch, h_gat, batch_2, pout):
  full = lambda a: pl.BlockSpec(a.shape, lambda: (0,) * a.ndim)
  args = (h_conv, batch.reshape(1, -1), h_gat, batch_2.reshape(1, -1),
          pout[0]['W'], pout[0]['b'].reshape(1, -1),
          pout[1]['W'], pout[1]['b'].reshape(1, -1),
          pout[2]['W'], pout[2]['b'].reshape(1, -1))
  return pl.pallas_call(
      _tail_body,
      in_specs=[full(a) for a in args],
      out_specs=pl.BlockSpec((NGRAPH, 1), lambda: (0, 0)),
      out_shape=jax.ShapeDtypeStruct((NGRAPH, 1), jnp.float32),
  )(*args)


def kernel(x, edge_attr, iso_type_2, params, edge_index, batch,
           assignment_index_2, edge_index_2, batch_2):
  src = jnp.pad(edge_index[0], (0, E_PAD - E_EDGES))
  dst = jnp.pad(edge_index[1], (0, E_PAD - E_EDGES), constant_values=N_NODES)
  ea_pad = jnp.pad(edge_attr, ((0, E_PAD - E_EDGES), (0, 0)))

  zeros16 = jnp.zeros((CH, 16), jnp.float32)
  zeros32 = jnp.zeros((CH, 32), jnp.float32)
  ones16 = jnp.ones((CH, 16), jnp.float32)

  h = x
  for p in params['conv']:
    mi, mo = p['root'].shape
    h_src = _sc_gather(h, src)
    msg = _tc_messages(ea_pad, h_src, p)
    aggp = _sc_scatter_sum(msg, dst, zeros32 if mo == 32 else zeros16)
    agg2 = aggp.reshape(NC, AGG_ROWS, mo)[:, :N_NODES, :]
    h = _tc_update(h, agg2, p)

  sums, cnts = _sc_assign_mean(h, assignment_index_2[0], assignment_index_2[1],
                               zeros32, ones16)
  wst = jnp.stack([g['W'] for g in params['gat']])
  a1s = jnp.stack([g['a1'] for g in params['gat']])
  a2s = jnp.stack([g['a2'] for g in params['gat']])
  wh = _tc_gat_prep(sums.reshape(NC, ISO_N, DIM),
                    cnts.reshape(NC, ISO_N, 16), iso_type_2, wst)

  adj_flat = _sc_adj(edge_index_2[0], edge_index_2[1],
                     jnp.zeros((16384,), jnp.float32),
                     jnp.ones((CH,), jnp.float32))
  adj = adj_flat[:ISO_N * ISO_N].reshape(ISO_N, ISO_N)

  h_gat = _tc_gat(adj, wh, a1s, a2s)
  z = _tc_tail(h, batch, h_gat, batch_2, params['out'])
  return z.reshape(-1)


# speedup vs baseline: 4.1110x; 4.1110x over previous
import functools

import jax
import jax.numpy as jnp
from jax import lax
from jax.experimental import pallas as pl
from jax.experimental.pallas import tpu as pltpu
from jax.experimental.pallas import tpu_sc as plsc

N_NODES = 10000
E_EDGES = 160000
D_FEAT = 32
D_EDGE = 16
DIM = 32
ISO_N = 4096
N_ASSIGN = 16384
E2 = 65536
NUM_I2 = 32
NHEADS = 4
NGRAPH = 128

NC = 2
NS = 16
NW = NC * NS
L = 16
CH = 128

E_PAD = ((E_EDGES + NW * CH - 1) // (NW * CH)) * (NW * CH)
AGG_ROWS = N_NODES + 16

_SC_PARAMS = pltpu.CompilerParams(use_tc_tiling_on_sc=False)
_SC_PARAMS_REG = pltpu.CompilerParams(use_tc_tiling_on_sc=False,
                                      needs_layout_passes=False)


def _sc_mesh():
  return plsc.VectorSubcoreMesh(core_axis_name="c", subcore_axis_name="s",
                                num_cores=NC, num_subcores=NS)


def _sc_gather(table, idx):
  V, D = table.shape
  B = idx.shape[0]
  n_ch = B // (NW * CH)

  @functools.partial(
      pl.kernel, mesh=_sc_mesh(), compiler_params=_SC_PARAMS,
      out_type=jax.ShapeDtypeStruct((B, D), jnp.float32),
      scratch_types=[
          pltpu.VMEM((CH,), jnp.int32),
          pltpu.VMEM((CH, D), jnp.float32),
          pltpu.SemaphoreType.DMA,
      ])
  def k(table_h, idx_h, out_h, idx_v, rows_v, sem):
    wid = lax.axis_index("s") * NC + lax.axis_index("c")

    def body(j, carry):
      off = (wid * n_ch + j) * CH
      pltpu.sync_copy(idx_h.at[pl.ds(off, CH)], idx_v)
      pltpu.async_copy(table_h.at[idx_v], rows_v, sem).wait()
      pltpu.sync_copy(rows_v, out_h.at[pl.ds(off, CH)])
      return carry

    lax.fori_loop(0, n_ch, body, 0)

  return k(table, idx)


def _sc_scatter_sum(msg, dst, zeros):
  B, D = msg.shape
  zch = zeros.shape[0]
  per_core = B // NC
  n_ch = per_core // (NS * CH)
  rows_pt = AGG_ROWS // NS

  @functools.partial(
      pl.kernel, mesh=_sc_mesh(), compiler_params=_SC_PARAMS,
      out_type=jax.ShapeDtypeStruct((NC * AGG_ROWS, D), jnp.float32),
      scratch_types=[
          pltpu.VMEM((CH,), jnp.int32),
          pltpu.VMEM((CH, D), jnp.float32),
          pltpu.VMEM((zch, D), jnp.float32),
          pltpu.VMEM_SHARED((AGG_ROWS, D), jnp.float32),
          pltpu.SemaphoreType.DMA,
      ])
  def k(msg_h, dst_h, zeros_h, out_h, idx_v, rows_v, zb_v, acc_s, sem):
    c = lax.axis_index("c")
    s = lax.axis_index("s")
    pltpu.sync_copy(zeros_h, zb_v)
    done = 0
    while done < rows_pt:
      m = min(zch, rows_pt - done)
      pltpu.sync_copy(zb_v.at[pl.ds(0, m)], acc_s.at[pl.ds(s * rows_pt + done, m)])
      done += m
    plsc.subcore_barrier()

    def body(j, carry):
      off = c * per_core + (s * n_ch + j) * CH
      pltpu.sync_copy(dst_h.at[pl.ds(off, CH)], idx_v)
      pltpu.sync_copy(msg_h.at[pl.ds(off, CH)], rows_v)
      pltpu.sync_copy(rows_v, acc_s.at[idx_v], add=True)
      return carry

    lax.fori_loop(0, n_ch, body, 0)
    plsc.subcore_barrier()
    done = 0
    while done < rows_pt:
      m = min(zch, rows_pt - done)
      r0 = s * rows_pt + done
      pltpu.sync_copy(acc_s.at[pl.ds(r0, m)], zb_v.at[pl.ds(0, m)])
      pltpu.sync_copy(zb_v.at[pl.ds(0, m)], out_h.at[pl.ds(c * AGG_ROWS + r0, m)])
      done += m

  return k(msg, dst, zeros)


def _sc_assign_mean(table, a_src, a_dst, zeros32, ones16):
  V, D = table.shape
  B = a_src.shape[0]
  per_core = B // NC
  n_ch = per_core // (NS * CH)
  rows_pt = ISO_N // NS
  zch = zeros32.shape[0]

  @functools.partial(
      pl.kernel, mesh=_sc_mesh(), compiler_params=_SC_PARAMS,
      out_type=(jax.ShapeDtypeStruct((NC * ISO_N, D), jnp.float32),
                jax.ShapeDtypeStruct((NC * ISO_N, 16), jnp.float32)),
      scratch_types=[
          pltpu.VMEM((CH,), jnp.int32),
          pltpu.VMEM((CH,), jnp.int32),
          pltpu.VMEM((CH, D), jnp.float32),
          pltpu.VMEM((CH, 16), jnp.float32),
          pltpu.VMEM((zch, D), jnp.float32),
          pltpu.VMEM_SHARED((ISO_N, D), jnp.float32),
          pltpu.VMEM_SHARED((ISO_N, 16), jnp.float32),
          pltpu.SemaphoreType.DMA,
      ])
  def k(table_h, src_h, dst_h, zeros_h, ones_h, sum_h, cnt_h,
        si_v, di_v, rows_v, ones_v, zb_v, sum_s, cnt_s, sem):
    c = lax.axis_index("c")
    s = lax.axis_index("s")
    pltpu.sync_copy(zeros_h, zb_v)
    pltpu.sync_copy(ones_h, ones_v)
    done = 0
    while done < rows_pt:
      m = min(zch, rows_pt - done)
      r0 = s * rows_pt + done
      pltpu.sync_copy(zb_v.at[pl.ds(0, m)], sum_s.at[pl.ds(r0, m)])
      pltpu.sync_copy(zb_v.at[pl.ds(0, m), pl.ds(0, 16)], cnt_s.at[pl.ds(r0, m)])
      done += m
    plsc.subcore_barrier()

    def body(j, carry):
      off = c * per_core + (s * n_ch + j) * CH
      pltpu.sync_copy(src_h.at[pl.ds(off, CH)], si_v)
      pltpu.sync_copy(dst_h.at[pl.ds(off, CH)], di_v)
      pltpu.async_copy(table_h.at[si_v], rows_v, sem).wait()
      pltpu.sync_copy(rows_v, sum_s.at[di_v], add=True)
      pltpu.sync_copy(ones_v, cnt_s.at[di_v], add=True)
      return carry

    lax.fori_loop(0, n_ch, body, 0)
    plsc.subcore_barrier()
    done = 0
    while done < rows_pt:
      m = min(zch, rows_pt - done)
      r0 = s * rows_pt + done
      pltpu.sync_copy(sum_s.at[pl.ds(r0, m)], zb_v.at[pl.ds(0, m)])
      pltpu.sync_copy(zb_v.at[pl.ds(0, m)], sum_h.at[pl.ds(c * ISO_N + r0, m)])
      pltpu.sync_copy(cnt_s.at[pl.ds(r0, m)], zb_v.at[pl.ds(0, m), pl.ds(0, 16)])
      pltpu.sync_copy(zb_v.at[pl.ds(0, m), pl.ds(0, 16)],
                      cnt_h.at[pl.ds(c * ISO_N + r0, m)])
      done += m

  return k(table, a_src, a_dst, zeros32, ones16)


ADJ_TOT = ISO_N * ISO_N + 64


def _sc_adj(e0, e1):
  SLAB_R = 16
  SLAB = SLAB_R * ISO_N
  n_pass = ISO_N // (SLAB_R * NW)
  CHE = 16384
  n_che = E2 // CHE

  @functools.partial(
      pl.kernel, mesh=_sc_mesh(), compiler_params=_SC_PARAMS_REG,
      out_type=jax.ShapeDtypeStruct((ISO_N * ISO_N,), jnp.float32),
      scratch_types=[
          pltpu.VMEM((CHE,), jnp.int32),
          pltpu.VMEM((CHE,), jnp.int32),
          pltpu.VMEM((SLAB,), jnp.float32),
      ])
  def k(e0_h, e1_h, out_h, e0_v, e1_v, slab_v):
    wid = lax.axis_index("s") * NC + lax.axis_index("c")
    zero = jnp.zeros((L,), jnp.float32)
    one = jnp.ones((L,), jnp.float32)

    def one_pass(p, carry):
      base_row = (wid * n_pass + p) * SLAB_R

      def zbody(i, cz):
        slab_v[pl.ds(i * L, L)] = zero
        return cz

      lax.fori_loop(0, SLAB // L, zbody, 0)

      def chunk(ci, cc):
        pltpu.sync_copy(e0_h.at[pl.ds(ci * CHE, CHE)], e0_v)
        pltpu.sync_copy(e1_h.at[pl.ds(ci * CHE, CHE)], e1_v)

        def vbody(i, cv):
          r = e0_v[pl.ds(i * L, L)] - base_row
          col = e1_v[pl.ds(i * L, L)]
          idx = r * ISO_N + col
          mask = (r >= 0) & (r < SLAB_R)
          plsc.store_scatter(slab_v, [idx], one, mask=mask)
          return cv

        lax.fori_loop(0, CHE // L, vbody, 0)
        return cc

      lax.fori_loop(0, n_che, chunk, 0)
      pltpu.sync_copy(slab_v, out_h.at[pl.ds(base_row * ISO_N, SLAB)])
      return carry

    lax.fori_loop(0, n_pass, one_pass, 0)

  return k(e0, e1)


_TC_PARAMS = pltpu.CompilerParams(dimension_semantics=("arbitrary",))


def _dot(a, b, dn=None, hi=False):
  if dn is None:
    dn = (((a.ndim - 1,), (0,)), ((), ()))
  prec = lax.Precision.HIGHEST if hi else None
  return lax.dot_general(a, b, dn, precision=prec,
                         preferred_element_type=jnp.float32)


def _msg_body(ea_ref, hs_ref, w1_ref, b1_ref, w2_ref, b2_ref, r_ref, s_ref,
              out_ref):
  e = jnp.maximum(_dot(ea_ref[...], w1_ref[...]) + b1_ref[...], 0.0)
  we = _dot(e, w2_ref[...]) + b2_ref[...]
  hx = _dot(hs_ref[...], r_ref[...], hi=True)
  out_ref[...] = _dot(we * hx, s_ref[...], hi=True)


def _tc_messages(ea_pad, h_src, p):
  mi, mo = p['root'].shape
  blk = 2048
  grid = ea_pad.shape[0] // blk
  rmat = jnp.kron(jnp.eye(mi, dtype=jnp.float32),
                  jnp.ones((1, mo), jnp.float32))
  smat = jnp.kron(jnp.ones((mi, 1), jnp.float32),
                  jnp.eye(mo, dtype=jnp.float32))
  full = lambda a: pl.BlockSpec(a.shape, lambda i: (0,) * a.ndim)
  args = (ea_pad, h_src, p['nn_W1'], p['nn_b1'].reshape(1, -1), p['nn_W2'],
          p['nn_b2'].reshape(1, -1), rmat, smat)
  return pl.pallas_call(
      _msg_body,
      grid=(grid,),
      in_specs=[pl.BlockSpec((blk, D_EDGE), lambda i: (i, 0)),
                pl.BlockSpec((blk, mi), lambda i: (i, 0))] +
               [full(a) for a in args[2:]],
      out_specs=pl.BlockSpec((blk, mo), lambda i: (i, 0)),
      out_shape=jax.ShapeDtypeStruct((ea_pad.shape[0], mo), jnp.float32),
      compiler_params=_TC_PARAMS,
  )(*args)


def _upd_body(h_ref, agg_ref, root_ref, bias_ref, out_ref):
  agg = agg_ref[0] + agg_ref[1]
  out_ref[...] = jnp.maximum(_dot(h_ref[...], root_ref[...]) + bias_ref[...]
                             + agg, 0.0)


def _tc_update(h, agg2, p):
  mi, mo = p['root'].shape
  full = lambda a: pl.BlockSpec(a.shape, lambda: (0,) * a.ndim)
  args = (h, agg2, p['root'], p['bias'].reshape(1, -1))
  return pl.pallas_call(
      _upd_body,
      in_specs=[full(a) for a in args],
      out_specs=pl.BlockSpec((N_NODES, mo), lambda: (0, 0)),
      out_shape=jax.ShapeDtypeStruct((N_NODES, mo), jnp.float32),
  )(*args)


def _prep_body(sum_ref, cnt_ref, iso_ref, w_ref, a1_ref, a2_ref, out_ref,
               f_ref):
  s = sum_ref[0] + sum_ref[1]
  c = cnt_ref[0] + cnt_ref[1]
  h2 = s / jnp.maximum(c[:, 0:1], 1.0)
  cat = jnp.concatenate([h2, iso_ref[...]], axis=1)
  for h in range(NHEADS):
    wh = _dot(cat, w_ref[h])
    out_ref[:, h * DIM:(h + 1) * DIM] = wh
    f_ref[:, h:h + 1] = jnp.sum(wh * a1_ref[h:h + 1, :], axis=1,
                                keepdims=True)
    f_ref[:, NHEADS + h:NHEADS + h + 1] = jnp.sum(
        wh * a2_ref[h:h + 1, :], axis=1, keepdims=True)


def _tc_gat_prep(sums, cnts, iso, wst, a1s, a2s):
  full = lambda a: pl.BlockSpec(a.shape, lambda: (0,) * a.ndim)
  args = (sums, cnts, iso, wst, a1s, a2s)
  return pl.pallas_call(
      _prep_body,
      in_specs=[full(a) for a in args],
      out_specs=[pl.BlockSpec((ISO_N, NHEADS * DIM), lambda: (0, 0)),
                 pl.BlockSpec((ISO_N, 2 * NHEADS), lambda: (0, 0))],
      out_shape=[jax.ShapeDtypeStruct((ISO_N, NHEADS * DIM), jnp.float32),
                 jax.ShapeDtypeStruct((ISO_N, 2 * NHEADS), jnp.float32)],
  )(*args)


def _gat_body(adj_ref, wh_ref, f_ref, ft_ref, out_ref):
  i = pl.program_id(0)
  blk = adj_ref.shape[0]
  mask = adj_ref[...] > 0.0
  acc = jnp.zeros((blk, DIM), jnp.float32)
  for h in range(NHEADS):
    whh = wh_ref[:, h * DIM:(h + 1) * DIM]
    f1 = f_ref[pl.ds(i * blk, blk), h:h + 1]
    f2t = ft_ref[NHEADS + h:NHEADS + h + 1, :]
    e = f1 + f2t
    e = jnp.where(e >= 0.0, e, 0.2 * e)
    e = jnp.where(mask, e, -9e15)
    m = jnp.max(e, axis=1, keepdims=True)
    pexp = jnp.exp(e - m)
    ssum = jnp.sum(pexp, axis=1, keepdims=True)
    acc = acc + _dot(pexp / ssum, whh)
  a = acc * (1.0 / NHEADS)
  out_ref[...] = jnp.where(a > 0.0, a, jnp.exp(a) - 1.0)


def _tc_gat(adj, wh, fmat, fmat_t):
  blk = 256
  grid = ISO_N // blk
  full = lambda a: pl.BlockSpec(a.shape, lambda i: (0,) * a.ndim)
  return pl.pallas_call(
      _gat_body,
      grid=(grid,),
      in_specs=[pl.BlockSpec((blk, ISO_N), lambda i: (i, 0)),
                full(wh), full(fmat), full(fmat_t)],
      out_specs=pl.BlockSpec((blk, DIM), lambda i: (i, 0)),
      out_shape=jax.ShapeDtypeStruct((ISO_N, DIM), jnp.float32),
      compiler_params=_TC_PARAMS,
  )(adj, wh, fmat, fmat_t)


def _seg_mean_mm(ids_row, data):
  gi = lax.broadcasted_iota(jnp.int32, (NGRAPH, 1), 0)
  m = (ids_row == gi).astype(jnp.float32)
  ssum = _dot(m, data, hi=True)
  cnt = jnp.sum(m, axis=1, keepdims=True)
  return ssum / jnp.maximum(cnt, 1.0)


def _tail_body(h_ref, b_ref, g_ref, b2_ref, w0_ref, c0_ref, w1_ref, c1_ref,
               w2_ref, c2_ref, out_ref):
  x1 = _seg_mean_mm(b_ref[...], h_ref[...])
  x2 = _seg_mean_mm(b2_ref[...], g_ref[...])
  z = jnp.concatenate([x1, x2], axis=1)
  z = jnp.maximum(_dot(z, w0_ref[...]) + c0_ref[...], 0.0)
  z = jnp.maximum(_dot(z, w1_ref[...]) + c1_ref[...], 0.0)
  out_ref[...] = _dot(z, w2_ref[...], hi=True) + c2_ref[...]


def _tc_tail(h_conv, batch, h_gat, batch_2, pout):
  full = lambda a: pl.BlockSpec(a.shape, lambda: (0,) * a.ndim)
  args = (h_conv, batch.reshape(1, -1), h_gat, batch_2.reshape(1, -1),
          pout[0]['W'], pout[0]['b'].reshape(1, -1),
          pout[1]['W'], pout[1]['b'].reshape(1, -1),
          pout[2]['W'], pout[2]['b'].reshape(1, -1))
  return pl.pallas_call(
      _tail_body,
      in_specs=[full(a) for a in args],
      out_specs=pl.BlockSpec((NGRAPH, 1), lambda: (0, 0)),
      out_shape=jax.ShapeDtypeStruct((NGRAPH, 1), jnp.float32),
  )(*args)


def kernel(x, edge_attr, iso_type_2, params, edge_index, batch,
           assignment_index_2, edge_index_2, batch_2):
  src = jnp.pad(edge_index[0], (0, E_PAD - E_EDGES))
  dst = jnp.pad(edge_index[1], (0, E_PAD - E_EDGES), constant_values=N_NODES)
  ea_pad = jnp.pad(edge_attr, ((0, E_PAD - E_EDGES), (0, 0)))

  zeros16 = jnp.zeros((CH, 16), jnp.float32)
  zeros32 = jnp.zeros((CH, 32), jnp.float32)
  ones16 = jnp.ones((CH, 16), jnp.float32)

  h = x
  for p in params['conv']:
    mi, mo = p['root'].shape
    h_src = _sc_gather(h, src)
    msg = _tc_messages(ea_pad, h_src, p)
    aggp = _sc_scatter_sum(msg, dst, zeros32 if mo == 32 else zeros16)
    agg2 = aggp.reshape(NC, AGG_ROWS, mo)[:, :N_NODES, :]
    h = _tc_update(h, agg2, p)

  sums, cnts = _sc_assign_mean(h, assignment_index_2[0], assignment_index_2[1],
                               zeros32, ones16)
  wst = jnp.stack([g['W'] for g in params['gat']])
  a1s = jnp.stack([g['a1'] for g in params['gat']])
  a2s = jnp.stack([g['a2'] for g in params['gat']])
  wh, fmat = _tc_gat_prep(sums.reshape(NC, ISO_N, DIM),
                          cnts.reshape(NC, ISO_N, 16), iso_type_2, wst,
                          a1s, a2s)

  adj = _sc_adj(edge_index_2[0], edge_index_2[1]).reshape(ISO_N, ISO_N)

  h_gat = _tc_gat(adj, wh, fmat, fmat.T)
  z = _tc_tail(h, batch, h_gat, batch_2, params['out'])
  return z.reshape(-1)
